# Initial kernel scaffold; baseline (speedup 1.0000x reference)
import jax
import jax.numpy as jnp
from jax.experimental import pallas as pl


def kernel(xyz1, code, W_dec, neighbors_src, neighbors_dst, edge_weights):
    raise NotImplementedError("write your pallas kernel here")


# speedup vs baseline: 366.4668x; 366.4668x over previous
import functools

import jax
import jax.numpy as jnp
from jax import lax
from jax.experimental import pallas as pl
from jax.experimental.pallas import tpu as pltpu
from jax.experimental.pallas import tpu_sc as plsc

NC = 2
NS = 16
LANES = 16

CHUNK = 128


def _decode_body(code_ref, w_ref, x_ref, y_ref):
    y_ref[...] = (
        jnp.dot(code_ref[...], w_ref[...], preferred_element_type=jnp.float32)
        - x_ref[...]
    )


def _decode_sub(code, w_dec, xflat, tm=2048):
    b, d = code.shape
    m = w_dec.shape[1]
    nb = pl.cdiv(m, tm)
    return pl.pallas_call(
        _decode_body,
        grid=(nb,),
        in_specs=[
            pl.BlockSpec((b, d), lambda j: (0, 0)),
            pl.BlockSpec((d, tm), lambda j: (0, j)),
            pl.BlockSpec((b, tm), lambda j: (0, j)),
        ],
        out_specs=pl.BlockSpec((b, tm), lambda j: (0, j)),
        out_shape=jax.ShapeDtypeStruct((b, m), jnp.float32),
    )(code, w_dec, xflat)


def _grad_body(nb, m, gscale, escale, g_ref, w_ref, ep_ref, out_ref, me_ref):
    j = pl.program_id(0)
    tm = w_ref.shape[1]

    @pl.when(j == 0)
    def _():
        out_ref[...] = jnp.zeros_like(out_ref)
        ep = ep_ref[...]
        col = lax.broadcasted_iota(jnp.int32, ep.shape, 1) % 8
        e0 = jnp.sum(jnp.where(col < 3, ep, 0.0))
        e1 = jnp.sum(jnp.where((col >= 3) & (col < 6), ep, 0.0))
        c2 = lax.broadcasted_iota(jnp.int32, (1, 2), 1)
        me_ref[...] = escale * (
            jnp.where(c2 == 0, e0, 0.0) + jnp.where(c2 == 1, e1, 0.0)
        )

    g = g_ref[0] + g_ref[1]
    w = w_ref[...]

    @pl.when(j == nb - 1)
    def _():
        col = j * tm + lax.broadcasted_iota(jnp.int32, (1, tm), 1)
        wm = jnp.where(col < m, w, 0.0)
        out_ref[...] += gscale * lax.dot_general(
            g, wm, (((1,), (1,)), ((), ())),
            preferred_element_type=jnp.float32,
        )

    @pl.when(j < nb - 1)
    def _():
        out_ref[...] += gscale * lax.dot_general(
            g, w, (((1,), (1,)), ((), ())),
            preferred_element_type=jnp.float32,
        )


def _grad_mm(gpad, w_dec, epart, gscale, escale, tm=2048):
    d, m = w_dec.shape
    b = gpad.shape[1]
    nb = gpad.shape[2] // tm
    return pl.pallas_call(
        functools.partial(_grad_body, nb, m, gscale, escale),
        grid=(nb,),
        in_specs=[
            pl.BlockSpec((NC, b, tm), lambda j: (0, 0, j)),
            pl.BlockSpec((d, tm), lambda j: (0, j)),
            pl.BlockSpec(epart.shape, lambda j: (0, 0)),
        ],
        out_specs=[
            pl.BlockSpec((b, d), lambda j: (0, 0)),
            pl.BlockSpec((1, 2), lambda j: (0, 0)),
        ],
        out_shape=[
            jax.ShapeDtypeStruct((b, d), jnp.float32),
            jax.ShapeDtypeStruct((1, 2), jnp.float32),
        ],
    )(gpad, w_dec, epart)


def _sc_edges_body(n, et, nch, y8_hbm, z_hbm, src_hbm, dst_hbm, w_hbm,
                   g8_hbm, ep_hbm, ysh, gsh, stage, sidx, didx, wbuf, yd, ys,
                   tb, tn, ebuf):
    cid = lax.axis_index("c")
    sid = lax.axis_index("s")
    wid = cid * NS + sid
    rpt = n // NS
    r0 = sid * rpt

    pltpu.sync_copy(y8_hbm.at[pl.ds(r0, rpt), :], stage)
    pltpu.sync_copy(stage, ysh.at[pl.ds(r0, rpt), :])
    pltpu.sync_copy(z_hbm.at[pl.ds(r0, rpt), :], stage)
    pltpu.sync_copy(stage, gsh.at[pl.ds(r0, rpt), :])
    plsc.subcore_barrier()

    ebase = wid * et
    lanes = lax.iota(jnp.int32, LANES)
    rowsel = lanes >> 3
    colsel = lanes & 7

    def chunk(i, eacc):
        off = ebase + i * CHUNK
        pltpu.sync_copy(src_hbm.at[pl.ds(off, CHUNK)], sidx)
        pltpu.sync_copy(dst_hbm.at[pl.ds(off, CHUNK)], didx)
        pltpu.sync_copy(w_hbm.at[pl.ds(off, CHUNK)], wbuf)
        pltpu.sync_copy(ysh.at[didx], yd)
        pltpu.sync_copy(ysh.at[sidx], ys)

        def vbody(jj, acc):
            rows = 2 * jj + rowsel
            wv = plsc.load_gather(wbuf, [rows])
            dv = (plsc.load_gather(yd, [rows, colsel])
                  - plsc.load_gather(ys, [rows, colsel]))
            tv = wv * dv
            plsc.store_scatter(tb, [rows, colsel], tv)
            plsc.store_scatter(tn, [rows, colsel], -tv)
            return acc + dv * tv

        eacc = lax.fori_loop(0, CHUNK // 2, vbody, eacc)
        pltpu.sync_copy(tb, gsh.at[didx], add=True)
        pltpu.sync_copy(tn, gsh.at[sidx], add=True)
        return eacc

    eacc = lax.fori_loop(0, nch, chunk, jnp.zeros((LANES,), jnp.float32))
    ebuf[...] = eacc
    pltpu.sync_copy(ebuf, ep_hbm.at[pl.ds(wid * LANES, LANES)])
    plsc.subcore_barrier()

    pltpu.sync_copy(gsh.at[pl.ds(r0, rpt), :], stage)
    pltpu.sync_copy(stage, g8_hbm.at[pl.ds(cid * n + r0, rpt), :])


def _sc_edges(y8, zeros8, srcp, dstp, wp, n, et, nch):
    mesh = plsc.VectorSubcoreMesh(core_axis_name="c", subcore_axis_name="s")
    body = functools.partial(_sc_edges_body, n, et, nch)
    fn = pl.kernel(
        body,
        out_type=(
            jax.ShapeDtypeStruct((NC * n, 8), jnp.float32),
            jax.ShapeDtypeStruct((NC * NS * LANES,), jnp.float32),
        ),
        mesh=mesh,
        compiler_params=pltpu.CompilerParams(
            needs_layout_passes=False, use_tc_tiling_on_sc=False),
        scratch_types=[
            pltpu.VMEM_SHARED((n, 8), jnp.float32),
            pltpu.VMEM_SHARED((n, 8), jnp.float32),
            pltpu.VMEM((n // NS, 8), jnp.float32),
            pltpu.VMEM((CHUNK,), jnp.int32),
            pltpu.VMEM((CHUNK,), jnp.int32),
            pltpu.VMEM((CHUNK,), jnp.float32),
            pltpu.VMEM((CHUNK, 8), jnp.float32),
            pltpu.VMEM((CHUNK, 8), jnp.float32),
            pltpu.VMEM((CHUNK, 8), jnp.float32),
            pltpu.VMEM((CHUNK, 8), jnp.float32),
            pltpu.VMEM((LANES,), jnp.float32),
        ],
    )
    return fn(y8, zeros8, srcp, dstp, wp)


def kernel(xyz1, code, W_dec, neighbors_src, neighbors_dst, edge_weights):
    b, n, _ = xyz1.shape
    d = code.shape[1]
    e = neighbors_src.shape[0]
    m = 3 * n
    assert b == 2

    xflat = xyz1.reshape(b, m)
    ynat = _decode_sub(code, W_dec, xflat)

    np_ = pl.cdiv(n, NS * 8) * NS * 8
    y8 = jnp.pad(
        ynat.reshape(b, n, 3).transpose(1, 0, 2).reshape(n, 3 * b),
        ((0, np_ - n), (0, 8 - 3 * b)),
    )

    per_tile = pl.cdiv(e, NC * NS * CHUNK) * CHUNK
    ep_total = per_tile * NC * NS
    pad = ep_total - e
    srcp = jnp.pad(neighbors_src, (0, pad))
    dstp = jnp.pad(neighbors_dst, (0, pad))
    wp = jnp.pad(edge_weights, (0, pad))
    zeros8 = jnp.zeros((np_, 8), jnp.float32)

    g8flat, epflat = _sc_edges(y8, zeros8, srcp, dstp, wp, np_,
                               per_tile, per_tile // CHUNK)

    g8 = g8flat.reshape(NC, np_, 8)[:, :n, :]
    gstack = jnp.stack(
        [g8[:, :, 3 * i:3 * i + 3].reshape(NC, m) for i in range(b)], axis=1
    )
    tm = 2048
    pm = pl.cdiv(m, tm) * tm
    gpad = jnp.pad(gstack, ((0, 0), (0, 0), (0, pm - m)))
    epart = epflat.reshape(NC * NS, LANES)

    code_grad, me = _grad_mm(gpad, W_dec, epart,
                             gscale=2.0 / m, escale=1.0 / m, tm=tm)
    return me.reshape(b), code_grad


# speedup vs baseline: 555.3048x; 1.5153x over previous
import functools

import jax
import jax.numpy as jnp
from jax import lax
from jax.experimental import pallas as pl
from jax.experimental.pallas import tpu as pltpu
from jax.experimental.pallas import tpu_sc as plsc

NC = 2
NS = 16
LANES = 16

CHUNK = 1024


def _decode_body(code_ref, w_ref, x_ref, y_ref):
    y_ref[...] = (
        jnp.dot(code_ref[...], w_ref[...], preferred_element_type=jnp.float32)
        - x_ref[...]
    )


def _decode_sub(code, w_dec, xflat, tm=2048):
    b, d = code.shape
    m = w_dec.shape[1]
    nb = pl.cdiv(m, tm)
    return pl.pallas_call(
        _decode_body,
        grid=(nb,),
        in_specs=[
            pl.BlockSpec((b, d), lambda j: (0, 0)),
            pl.BlockSpec((d, tm), lambda j: (0, j)),
            pl.BlockSpec((b, tm), lambda j: (0, j)),
        ],
        out_specs=pl.BlockSpec((b, tm), lambda j: (0, j)),
        out_shape=jax.ShapeDtypeStruct((b, m), jnp.float32),
    )(code, w_dec, xflat)


def _grad_body(nb, m, gscale, escale, g_ref, w_ref, ep_ref, out_ref, me_ref):
    j = pl.program_id(0)
    tm = w_ref.shape[1]

    @pl.when(j == 0)
    def _():
        out_ref[...] = jnp.zeros_like(out_ref)
        ep = ep_ref[...]
        col = lax.broadcasted_iota(jnp.int32, ep.shape, 1) % 8
        e0 = jnp.sum(jnp.where(col < 3, ep, 0.0))
        e1 = jnp.sum(jnp.where((col >= 3) & (col < 6), ep, 0.0))
        c2 = lax.broadcasted_iota(jnp.int32, (1, 2), 1)
        me_ref[...] = escale * (
            jnp.where(c2 == 0, e0, 0.0) + jnp.where(c2 == 1, e1, 0.0)
        )

    g = g_ref[0] + g_ref[1]
    w = w_ref[...]

    @pl.when(j == nb - 1)
    def _():
        col = j * tm + lax.broadcasted_iota(jnp.int32, (1, tm), 1)
        wm = jnp.where(col < m, w, 0.0)
        out_ref[...] += gscale * lax.dot_general(
            g, wm, (((1,), (1,)), ((), ())),
            preferred_element_type=jnp.float32,
        )

    @pl.when(j < nb - 1)
    def _():
        out_ref[...] += gscale * lax.dot_general(
            g, w, (((1,), (1,)), ((), ())),
            preferred_element_type=jnp.float32,
        )


def _grad_mm(gpad, w_dec, epart, gscale, escale, tm=2048):
    d, m = w_dec.shape
    b = gpad.shape[1]
    nb = gpad.shape[2] // tm
    return pl.pallas_call(
        functools.partial(_grad_body, nb, m, gscale, escale),
        grid=(nb,),
        in_specs=[
            pl.BlockSpec((NC, b, tm), lambda j: (0, 0, j)),
            pl.BlockSpec((d, tm), lambda j: (0, j)),
            pl.BlockSpec(epart.shape, lambda j: (0, 0)),
        ],
        out_specs=[
            pl.BlockSpec((b, d), lambda j: (0, 0)),
            pl.BlockSpec((1, 2), lambda j: (0, 0)),
        ],
        out_shape=[
            jax.ShapeDtypeStruct((b, d), jnp.float32),
            jax.ShapeDtypeStruct((1, 2), jnp.float32),
        ],
    )(gpad, w_dec, epart)


def _sc_edges_body(n, et, nch, y8_hbm, z_hbm, src_hbm, dst_hbm, w_hbm,
                   g8_hbm, ep_hbm, ysh, gsh, stage, sidx, didx, wbuf, yd, ys,
                   tb, tn, ebuf):
    cid = lax.axis_index("c")
    sid = lax.axis_index("s")
    wid = cid * NS + sid
    rpt = n // NS
    r0 = sid * rpt

    pltpu.sync_copy(y8_hbm.at[pl.ds(r0, rpt), :], stage)
    pltpu.sync_copy(stage, ysh.at[pl.ds(r0, rpt), :])
    pltpu.sync_copy(z_hbm.at[pl.ds(r0, rpt), :], stage)
    pltpu.sync_copy(stage, gsh.at[pl.ds(r0, rpt), :])
    plsc.subcore_barrier()

    ebase = wid * et
    lanes = lax.iota(jnp.int32, LANES)
    rowsel = lanes >> 3
    colsel = lanes & 7

    def chunk(i, eacc):
        off = ebase + i * CHUNK
        pltpu.sync_copy(src_hbm.at[pl.ds(off, CHUNK)], sidx)
        pltpu.sync_copy(dst_hbm.at[pl.ds(off, CHUNK)], didx)
        pltpu.sync_copy(w_hbm.at[pl.ds(off, CHUNK)], wbuf)
        pltpu.sync_copy(ysh.at[didx], yd)
        pltpu.sync_copy(ysh.at[sidx], ys)

        def vbody(jj, acc):
            rows = 2 * jj + rowsel
            wv = plsc.load_gather(wbuf, [rows])
            dv = (plsc.load_gather(yd, [rows, colsel])
                  - plsc.load_gather(ys, [rows, colsel]))
            tv = wv * dv
            plsc.store_scatter(tb, [rows, colsel], tv)
            plsc.store_scatter(tn, [rows, colsel], -tv)
            return acc + dv * tv

        eacc = lax.fori_loop(0, CHUNK // 2, vbody, eacc)
        pltpu.sync_copy(tb, gsh.at[didx], add=True)
        pltpu.sync_copy(tn, gsh.at[sidx], add=True)
        return eacc

    eacc = lax.fori_loop(0, nch, chunk, jnp.zeros((LANES,), jnp.float32))
    ebuf[...] = eacc
    pltpu.sync_copy(ebuf, ep_hbm.at[pl.ds(wid * LANES, LANES)])
    plsc.subcore_barrier()

    pltpu.sync_copy(gsh.at[pl.ds(r0, rpt), :], stage)
    pltpu.sync_copy(stage, g8_hbm.at[pl.ds(cid * n + r0, rpt), :])


def _sc_edges(y8, zeros8, srcp, dstp, wp, n, et, nch):
    mesh = plsc.VectorSubcoreMesh(core_axis_name="c", subcore_axis_name="s")
    body = functools.partial(_sc_edges_body, n, et, nch)
    fn = pl.kernel(
        body,
        out_type=(
            jax.ShapeDtypeStruct((NC * n, 8), jnp.float32),
            jax.ShapeDtypeStruct((NC * NS * LANES,), jnp.float32),
        ),
        mesh=mesh,
        compiler_params=pltpu.CompilerParams(
            needs_layout_passes=False, use_tc_tiling_on_sc=False),
        scratch_types=[
            pltpu.VMEM_SHARED((n, 8), jnp.float32),
            pltpu.VMEM_SHARED((n, 8), jnp.float32),
            pltpu.VMEM((n // NS, 8), jnp.float32),
            pltpu.VMEM((CHUNK,), jnp.int32),
            pltpu.VMEM((CHUNK,), jnp.int32),
            pltpu.VMEM((CHUNK,), jnp.float32),
            pltpu.VMEM((CHUNK, 8), jnp.float32),
            pltpu.VMEM((CHUNK, 8), jnp.float32),
            pltpu.VMEM((CHUNK, 8), jnp.float32),
            pltpu.VMEM((CHUNK, 8), jnp.float32),
            pltpu.VMEM((LANES,), jnp.float32),
        ],
    )
    return fn(y8, zeros8, srcp, dstp, wp)


def kernel(xyz1, code, W_dec, neighbors_src, neighbors_dst, edge_weights):
    b, n, _ = xyz1.shape
    d = code.shape[1]
    e = neighbors_src.shape[0]
    m = 3 * n
    assert b == 2

    xflat = xyz1.reshape(b, m)
    ynat = _decode_sub(code, W_dec, xflat)

    np_ = pl.cdiv(n, NS * 8) * NS * 8
    y8 = jnp.pad(
        ynat.reshape(b, n, 3).transpose(1, 0, 2).reshape(n, 3 * b),
        ((0, np_ - n), (0, 8 - 3 * b)),
    )

    per_tile = pl.cdiv(e, NC * NS * CHUNK) * CHUNK
    ep_total = per_tile * NC * NS
    pad = ep_total - e
    srcp = jnp.pad(neighbors_src, (0, pad))
    dstp = jnp.pad(neighbors_dst, (0, pad))
    wp = jnp.pad(edge_weights, (0, pad))
    zeros8 = jnp.zeros((np_, 8), jnp.float32)

    g8flat, epflat = _sc_edges(y8, zeros8, srcp, dstp, wp, np_,
                               per_tile, per_tile // CHUNK)

    g8 = g8flat.reshape(NC, np_, 8)[:, :n, :]
    gstack = jnp.stack(
        [g8[:, :, 3 * i:3 * i + 3].reshape(NC, m) for i in range(b)], axis=1
    )
    tm = 2048
    pm = pl.cdiv(m, tm) * tm
    gpad = jnp.pad(gstack, ((0, 0), (0, 0), (0, pm - m)))
    epart = epflat.reshape(NC * NS, LANES)

    code_grad, me = _grad_mm(gpad, W_dec, epart,
                             gscale=2.0 / m, escale=1.0 / m, tm=tm)
    return me.reshape(b), code_grad


# speedup vs baseline: 739.5015x; 1.3317x over previous
import functools

import jax
import jax.numpy as jnp
from jax import lax
from jax.experimental import pallas as pl
from jax.experimental.pallas import tpu as pltpu
from jax.experimental.pallas import tpu_sc as plsc

NC = 2
NS = 16
LANES = 16

CHUNK = 1024
RPASS = 4


def _decode_body(code_ref, wt_ref, y_ref):
    y_ref[...] = lax.dot_general(
        code_ref[...], wt_ref[...], (((1,), (1,)), ((), ())),
        preferred_element_type=jnp.float32,
    )


def _decode(code, w_t, m2, tm=2048):
    b, d = code.shape
    nb = pl.cdiv(m2, tm)
    return pl.pallas_call(
        _decode_body,
        grid=(nb,),
        in_specs=[
            pl.BlockSpec((b, d), lambda j: (0, 0)),
            pl.BlockSpec((tm, d), lambda j: (j, 0)),
        ],
        out_specs=pl.BlockSpec((b, tm), lambda j: (0, j)),
        out_shape=jax.ShapeDtypeStruct((b, m2), jnp.float32),
    )(code, w_t)


def _grad_body(nb, m, gscale, escale, g_ref, wt_ref, ep_ref, out_ref, me_ref):
    j = pl.program_id(0)
    tm = wt_ref.shape[0]

    @pl.when(j == 0)
    def _():
        out_ref[...] = jnp.zeros_like(out_ref)
        ep = ep_ref[...]
        col = lax.broadcasted_iota(jnp.int32, ep.shape, 1) % 8
        e0 = jnp.sum(jnp.where(col < 3, ep, 0.0))
        e1 = jnp.sum(jnp.where((col >= 3) & (col < 6), ep, 0.0))
        c2 = lax.broadcasted_iota(jnp.int32, (1, 2), 1)
        me_ref[...] = escale * (
            jnp.where(c2 == 0, e0, 0.0) + jnp.where(c2 == 1, e1, 0.0)
        )

    g = g_ref[0] + g_ref[1]
    wt = wt_ref[...]

    @pl.when(j == nb - 1)
    def _():
        col = j * tm + lax.broadcasted_iota(jnp.int32, (1, tm), 1)
        gm = jnp.where(col < m, g, 0.0)
        colv = j * tm + lax.broadcasted_iota(jnp.int32, (tm, 1), 0)
        wm = jnp.where(colv < m, wt, 0.0)
        out_ref[...] += gscale * lax.dot_general(
            gm, wm, (((1,), (0,)), ((), ())),
            preferred_element_type=jnp.float32,
        )

    @pl.when(j < nb - 1)
    def _():
        out_ref[...] += gscale * lax.dot_general(
            g, wt, (((1,), (0,)), ((), ())),
            preferred_element_type=jnp.float32,
        )


def _grad_mm(gstack, w_t, epart, gscale, escale, tm=2048):
    m, d = w_t.shape
    b = gstack.shape[1]
    nb = pl.cdiv(m, tm)
    return pl.pallas_call(
        functools.partial(_grad_body, nb, m, gscale, escale),
        grid=(nb,),
        in_specs=[
            pl.BlockSpec((NC, b, tm), lambda j: (0, 0, j)),
            pl.BlockSpec((tm, d), lambda j: (j, 0)),
            pl.BlockSpec(epart.shape, lambda j: (0, 0)),
        ],
        out_specs=[
            pl.BlockSpec((b, d), lambda j: (0, 0)),
            pl.BlockSpec((1, 2), lambda j: (0, 0)),
        ],
        out_shape=[
            jax.ShapeDtypeStruct((b, d), jnp.float32),
            jax.ShapeDtypeStruct((1, 2), jnp.float32),
        ],
    )(gstack, w_t, epart)


def _sc_edges_body(n, et, nch, r_hbm, x_hbm, z_hbm, src_hbm, dst_hbm, w_hbm,
                   g8_hbm, ep_hbm, ysh, gsh, stage, rb0, rb1, xbuf,
                   sidx, didx, wbuf, yd, ys, tb, tn, ebuf):
    cid = lax.axis_index("c")
    sid = lax.axis_index("s")
    wid = cid * NS + sid
    rpt = n // NS
    r0 = sid * rpt
    lanes = lax.iota(jnp.int32, LANES)

    rq = rpt // RPASS
    zero16 = jnp.zeros((LANES,), jnp.float32)
    for p in range(RPASS):
        v0 = r0 + p * rq
        pltpu.sync_copy(r_hbm.at[0, pl.ds(3 * v0, 3 * rq)], rb0)
        pltpu.sync_copy(r_hbm.at[1, pl.ds(3 * v0, 3 * rq)], rb1)
        for k in range(3):
            for b2 in range(2):
                pltpu.sync_copy(x_hbm.at[k, b2, pl.ds(v0, rq)],
                                xbuf.at[k * 2 + b2])
        for k in range(3):
            for b2 in range(2):
                rb = rb0 if b2 == 0 else rb1
                colv = jnp.full((LANES,), 3 * b2 + k, jnp.int32)

                def pbody(jj, _, rb=rb, colv=colv, k=k, pk=k * 2 + b2):
                    rows = jj * LANES + lanes
                    vr = plsc.load_gather(rb, [3 * rows + k])
                    xv = xbuf[pk, pl.ds(jj * LANES, LANES)]
                    plsc.store_scatter(stage, [rows, colv], vr - xv)
                    return 0

                lax.fori_loop(0, rq // LANES, pbody, 0)
        for c78 in (6, 7):
            colv = jnp.full((LANES,), c78, jnp.int32)

            def zbody(jj, _, colv=colv):
                rows = jj * LANES + lanes
                plsc.store_scatter(stage, [rows, colv], zero16)
                return 0

            lax.fori_loop(0, rq // LANES, zbody, 0)

        pltpu.sync_copy(stage, ysh.at[pl.ds(v0, rq), :])
        pltpu.sync_copy(z_hbm.at[pl.ds(v0, rq), :], stage)
        pltpu.sync_copy(stage, gsh.at[pl.ds(v0, rq), :])
    plsc.subcore_barrier()

    ebase = wid * et
    rowsel = lanes >> 3
    colsel = lanes & 7

    def chunk(i, eacc):
        off = ebase + i * CHUNK
        pltpu.sync_copy(src_hbm.at[pl.ds(off, CHUNK)], sidx)
        pltpu.sync_copy(dst_hbm.at[pl.ds(off, CHUNK)], didx)
        pltpu.sync_copy(w_hbm.at[pl.ds(off, CHUNK)], wbuf)
        pltpu.sync_copy(ysh.at[didx], yd)
        pltpu.sync_copy(ysh.at[sidx], ys)

        def vbody(jj, acc):
            rows = 2 * jj + rowsel
            wv = plsc.load_gather(wbuf, [rows])
            dv = (plsc.load_gather(yd, [rows, colsel])
                  - plsc.load_gather(ys, [rows, colsel]))
            tv = wv * dv
            plsc.store_scatter(tb, [rows, colsel], tv)
            plsc.store_scatter(tn, [rows, colsel], -tv)
            return acc + dv * tv

        eacc = lax.fori_loop(0, CHUNK // 2, vbody, eacc)
        pltpu.sync_copy(tb, gsh.at[didx], add=True)
        pltpu.sync_copy(tn, gsh.at[sidx], add=True)
        return eacc

    eacc = lax.fori_loop(0, nch, chunk, jnp.zeros((LANES,), jnp.float32))
    ebuf[...] = eacc
    pltpu.sync_copy(ebuf, ep_hbm.at[pl.ds(wid * LANES, LANES)])
    plsc.subcore_barrier()

    for p in range(RPASS):
        v0 = r0 + p * rq
        pltpu.sync_copy(gsh.at[pl.ds(v0, rq), :], stage)
        pltpu.sync_copy(stage, g8_hbm.at[pl.ds(cid * n + v0, rq), :])


def _sc_edges(rfull, xpl, zeros8, srcp, dstp, wp, n, et, nch):
    mesh = plsc.VectorSubcoreMesh(core_axis_name="c", subcore_axis_name="s")
    body = functools.partial(_sc_edges_body, n, et, nch)
    rpt = n // NS
    fn = pl.kernel(
        body,
        out_type=(
            jax.ShapeDtypeStruct((NC * n, 8), jnp.float32),
            jax.ShapeDtypeStruct((NC * NS * LANES,), jnp.float32),
        ),
        mesh=mesh,
        compiler_params=pltpu.CompilerParams(
            needs_layout_passes=False, use_tc_tiling_on_sc=False),
        scratch_types=[
            pltpu.VMEM_SHARED((n, 8), jnp.float32),
            pltpu.VMEM_SHARED((n, 8), jnp.float32),
            pltpu.VMEM((rpt // RPASS, 8), jnp.float32),
            pltpu.VMEM((3 * rpt // RPASS,), jnp.float32),
            pltpu.VMEM((3 * rpt // RPASS,), jnp.float32),
            pltpu.VMEM((6, rpt // RPASS), jnp.float32),
            pltpu.VMEM((CHUNK,), jnp.int32),
            pltpu.VMEM((CHUNK,), jnp.int32),
            pltpu.VMEM((CHUNK,), jnp.float32),
            pltpu.VMEM((CHUNK, 8), jnp.float32),
            pltpu.VMEM((CHUNK, 8), jnp.float32),
            pltpu.VMEM((CHUNK, 8), jnp.float32),
            pltpu.VMEM((CHUNK, 8), jnp.float32),
            pltpu.VMEM((LANES,), jnp.float32),
        ],
    )
    return fn(rfull, xpl, zeros8, srcp, dstp, wp)


def kernel(xyz1, code, W_dec, neighbors_src, neighbors_dst, edge_weights):
    b, n, _ = xyz1.shape
    d = code.shape[1]
    e = neighbors_src.shape[0]
    m = 3 * n
    assert b == 2

    w_t = W_dec.T
    np_ = pl.cdiv(n, NS * LANES) * NS * LANES
    m2 = 3 * np_
    rfull = _decode(code, w_t, m2)
    xpl = jnp.pad(xyz1.transpose(2, 0, 1), ((0, 0), (0, 0), (0, np_ - n)))

    per_tile = pl.cdiv(e, NC * NS * CHUNK) * CHUNK
    ep_total = per_tile * NC * NS
    pad = ep_total - e
    srcp = jnp.pad(neighbors_src, (0, pad))
    dstp = jnp.pad(neighbors_dst, (0, pad))
    wp = jnp.pad(edge_weights, (0, pad))
    zeros8 = jnp.zeros((np_, 8), jnp.float32)

    g8flat, epflat = _sc_edges(rfull, xpl, zeros8, srcp, dstp, wp, np_,
                               per_tile, per_tile // CHUNK)

    g8 = g8flat.reshape(NC, np_, 8)[:, :n, :]
    gstack = jnp.stack(
        [g8[:, :, 3 * i:3 * i + 3].reshape(NC, m) for i in range(b)], axis=1
    )
    epart = epflat.reshape(NC * NS, LANES)

    code_grad, me = _grad_mm(gstack, w_t, epart,
                             gscale=2.0 / m, escale=1.0 / m, tm=2048)
    return me.reshape(b), code_grad
